# baseline (speedup 1.0000x reference)
import functools

import jax
import jax.numpy as jnp
from jax import lax
from jax.experimental import pallas as pl
from jax.experimental.pallas import tpu as pltpu
from jax.experimental.pallas import tpu_sc as plsc


def _moe_body(base_ref, nblk_ref, blora_ref, x_ref, wgu_ref, wd_ref,
              gua_ref, gub_ref, da_ref, db_ref, rw_ref, rtv_ref, ys_ref,
              *, B, F, FC):
    e = pl.program_id(0)

    nt = (((1,), (1,)), ((), ()))
    f32 = jnp.float32
    D = x_ref.shape[1]
    R = gua_ref.shape[2]

    def block_body(j, carry):
        b = base_ref[e] + j
        l = blora_ref[b]

        rtv = rtv_ref[b, 0, :]
        iota_t = jax.lax.broadcasted_iota(jnp.int32, (B, x_ref.shape[0]), 1)
        sel = (rtv[:, None] == iota_t).astype(f32)
        xs = jnp.dot(sel, x_ref[...], preferred_element_type=f32)
        gua = gua_ref[l, 0]
        gub = gub_ref[l, 0]
        da = da_ref[l, 0]
        db = db_ref[l, 0]
        wd = wd_ref[0]

        u = jax.lax.dot_general(xs, gua, nt, preferred_element_type=f32)

        dn = jnp.zeros((B, D), f32)
        v = jnp.zeros((B, R), f32)
        for f0 in range(0, F, FC):
            wg = wgu_ref[0, f0:f0 + FC, :]
            wu = wgu_ref[0, F + f0:F + f0 + FC, :]
            gate = jax.lax.dot_general(xs, wg, nt, preferred_element_type=f32)
            gate += jnp.dot(u, gub[:, f0:f0 + FC],
                            preferred_element_type=f32)
            up = jax.lax.dot_general(xs, wu, nt, preferred_element_type=f32)
            up += jnp.dot(u, gub[:, F + f0:F + f0 + FC],
                          preferred_element_type=f32)
            act = gate / (1.0 + jnp.exp(-gate)) * up
            dn += jax.lax.dot_general(act, wd[:, f0:f0 + FC], nt,
                                      preferred_element_type=f32)
            v += jax.lax.dot_general(act, da[:, f0:f0 + FC], nt,
                                     preferred_element_type=f32)
        dn += jnp.dot(v, db, preferred_element_type=f32)
        ys_ref[b] = (dn * rw_ref[b]).astype(jnp.bfloat16)
        return carry

    jax.lax.fori_loop(0, nblk_ref[e], block_body, 0)


def kernel(hidden_states, topk_weights, w_gate_up, w_down, gate_up_lora_a,
           gate_up_lora_b, down_lora_a, down_lora_b, topk_ids,
           token_lora_ids):
    T, D = hidden_states.shape
    E, two_f, _ = w_gate_up.shape
    F = two_f // 2
    L, _, R, _ = gate_up_lora_a.shape
    K = topk_ids.shape[1]
    TK = T * K
    B = 128
    FC = 512
    NG = E * L
    NB = TK // B + NG

    i32 = jnp.int32
    tw = topk_weights / jnp.sum(topk_weights, axis=-1, keepdims=True)
    flat_w = tw.reshape(-1)
    flat_e = topk_ids.reshape(-1).astype(i32)
    flat_l = jnp.broadcast_to(token_lora_ids.astype(i32)[:, None],
                              (T, K)).reshape(-1)
    g = flat_e * L + flat_l

    onehot = (g[:, None] == jnp.arange(NG, dtype=i32)[None, :]).astype(i32)
    csum = jnp.cumsum(onehot, axis=0)
    counts = csum[-1]
    rank = jnp.take_along_axis(csum, g[:, None], axis=1)[:, 0] - 1

    bpg = (counts + B - 1) // B
    block_off = jnp.concatenate(
        [jnp.zeros((1,), i32), jnp.cumsum(bpg)[:-1].astype(i32)])
    bids = jnp.arange(NB, dtype=i32)
    bg = jnp.clip(jnp.searchsorted(block_off, bids, side='right').astype(i32)
                  - 1, 0, NG - 1)
    blora = bg % L

    bpe = bpg.reshape(E, L).sum(axis=1)
    base = jnp.concatenate(
        [jnp.zeros((1,), i32), jnp.cumsum(bpe)[:-1].astype(i32)])
    nblk = bpe.astype(i32)

    pslot = (block_off[g] + rank // B) * B + rank % B
    flat_t = jnp.arange(TK, dtype=i32) // K
    rt = jnp.zeros((NB * B,), i32).at[pslot].set(flat_t)
    rw = jnp.zeros((NB * B,), jnp.float32).at[pslot].set(flat_w)
    rw = rw.reshape(NB, B, 1)

    q = pslot.reshape(T, K)
    grid_spec = pltpu.PrefetchScalarGridSpec(
        num_scalar_prefetch=3,
        grid=(E,),
        in_specs=[
            pl.BlockSpec((T, D), lambda e, *s: (0, 0)),
            pl.BlockSpec((1, two_f, D), lambda e, *s: (e, 0, 0)),
            pl.BlockSpec((1, D, F), lambda e, *s: (e, 0, 0)),
            pl.BlockSpec((L, 1, R, D), lambda e, *s: (0, e, 0, 0)),
            pl.BlockSpec((L, 1, R, two_f), lambda e, *s: (0, e, 0, 0)),
            pl.BlockSpec((L, 1, R, F), lambda e, *s: (0, e, 0, 0)),
            pl.BlockSpec((L, 1, R, D), lambda e, *s: (0, e, 0, 0)),
            pl.BlockSpec((NB, B, 1), lambda e, *s: (0, 0, 0)),
            pl.BlockSpec((NB, 1, B), lambda e, *s: (0, 0, 0)),
        ],
        out_specs=pl.BlockSpec((NB, B, D), lambda e, *s: (0, 0, 0)),
        scratch_shapes=[],
    )
    ys = pl.pallas_call(
        functools.partial(_moe_body, B=B, F=F, FC=FC),
        grid_spec=grid_spec,
        out_shape=jax.ShapeDtypeStruct((NB, B, D), jnp.bfloat16),
        compiler_params=pltpu.CompilerParams(
            vmem_limit_bytes=100 * 1024 * 1024),
    )(base, nblk, blora, hidden_states, w_gate_up, w_down,
      gate_up_lora_a, gate_up_lora_b.transpose(0, 1, 3, 2), down_lora_a,
      down_lora_b.transpose(0, 1, 3, 2), rw, rt.reshape(NB, 1, B))

    ys32 = jax.lax.bitcast_convert_type(
        ys.reshape(NB * B, D // 2, 2), jnp.int32)
    g0, g1 = _sc_combine(ys32, q[:, 0], q[:, 1])
    b0 = jax.lax.bitcast_convert_type(g0[..., None],
                                      jnp.bfloat16).reshape(T, D)
    b1 = jax.lax.bitcast_convert_type(g1[..., None],
                                      jnp.bfloat16).reshape(T, D)
    return b0.astype(jnp.float32) + b1.astype(jnp.float32)


def _sc_combine(ys, q0, q1):
    TT, DW = q0.shape[0], ys.shape[1]
    NC, NS = 2, 16
    NW = NC * NS
    TPW = TT // NW
    CH = 32
    LW = 16
    mesh = plsc.VectorSubcoreMesh(core_axis_name="c", subcore_axis_name="s")

    @functools.partial(
        pl.kernel, mesh=mesh,
        out_type=(jax.ShapeDtypeStruct((TT, DW), jnp.int32),
                  jax.ShapeDtypeStruct((TT, DW), jnp.int32)),
        scratch_types=[
            pltpu.VMEM((CH,), jnp.int32),
            pltpu.VMEM((CH,), jnp.int32),
            pltpu.VMEM((CH, DW), jnp.int32),
            pltpu.VMEM((CH, DW), jnp.int32),
            pltpu.SemaphoreType.DMA,
            pltpu.SemaphoreType.DMA,
        ],
    )
    def combine(ys_hbm, q0_hbm, q1_hbm, g0_hbm, g1_hbm, i0_v, i1_v, r0_v,
                r1_v, s0, s1):
        wid = lax.axis_index("s") * NC + lax.axis_index("c")
        for c in range(TPW // CH):
            base = wid * TPW + c * CH
            pltpu.sync_copy(q0_hbm.at[pl.ds(base, CH)], i0_v)
            pltpu.sync_copy(q1_hbm.at[pl.ds(base, CH)], i1_v)
            cp0 = pltpu.async_copy(ys_hbm.at[i0_v], r0_v, s0)
            cp1 = pltpu.async_copy(ys_hbm.at[i1_v], r1_v, s1)
            cp0.wait()
            cp1.wait()
            pltpu.sync_copy(r0_v, g0_hbm.at[pl.ds(base, CH)])
            pltpu.sync_copy(r1_v, g1_hbm.at[pl.ds(base, CH)])

    return combine(ys, q0, q1)


# speedup vs baseline: 1.4774x; 1.4774x over previous
import functools

import jax
import jax.numpy as jnp
from jax.experimental import pallas as pl
from jax.experimental.pallas import tpu as pltpu


def _moe_body(base_ref, nblk_ref, blora_ref, rt_ref, x_ref, wgu_ref, wd_ref,
              gua_ref, gub_ref, da_ref, db_ref, rw_ref, rtv_ref, out_ref,
              dn_ref, *, B, F, FC):
    e = pl.program_id(0)

    @pl.when(e == 0)
    def _init():
        out_ref[...] = jnp.zeros_like(out_ref)

    nt = (((1,), (1,)), ((), ()))
    f32 = jnp.float32
    D = x_ref.shape[1]
    R = gua_ref.shape[2]

    def block_body(j, carry):
        b = base_ref[e] + j
        l = blora_ref[b]

        rtv = rtv_ref[b, 0, :]
        iota_t = jax.lax.broadcasted_iota(jnp.int32, (B, x_ref.shape[0]), 1)
        sel = (rtv[:, None] == iota_t).astype(f32)
        xs = jnp.dot(sel, x_ref[...], preferred_element_type=f32)
        gua = gua_ref[l, 0]
        gub = gub_ref[l, 0]
        da = da_ref[l, 0]
        db = db_ref[l, 0]
        wd = wd_ref[0]

        u = jax.lax.dot_general(xs, gua, nt, preferred_element_type=f32)

        dn = jnp.zeros((B, D), f32)
        v = jnp.zeros((B, R), f32)
        for f0 in range(0, F, FC):
            wg = wgu_ref[0, f0:f0 + FC, :]
            wu = wgu_ref[0, F + f0:F + f0 + FC, :]
            gate = jax.lax.dot_general(xs, wg, nt, preferred_element_type=f32)
            gate += jnp.dot(u, gub[:, f0:f0 + FC],
                            preferred_element_type=f32)
            up = jax.lax.dot_general(xs, wu, nt, preferred_element_type=f32)
            up += jnp.dot(u, gub[:, F + f0:F + f0 + FC],
                          preferred_element_type=f32)
            act = gate / (1.0 + jnp.exp(-gate)) * up
            dn += jax.lax.dot_general(act, wd[:, f0:f0 + FC], nt,
                                      preferred_element_type=f32)
            v += jax.lax.dot_general(act, da[:, f0:f0 + FC], nt,
                                     preferred_element_type=f32)
        dn += jnp.dot(v, db, preferred_element_type=f32)
        dn_ref[...] = dn * rw_ref[b]

        def scatter_body(r, c):
            t = rt_ref[b * B + r]
            out_ref[t, :] = out_ref[t, :] + dn_ref[r, :]
            return c

        jax.lax.fori_loop(0, B, scatter_body, 0, unroll=8)
        return carry

    jax.lax.fori_loop(0, nblk_ref[e], block_body, 0)


def kernel(hidden_states, topk_weights, w_gate_up, w_down, gate_up_lora_a,
           gate_up_lora_b, down_lora_a, down_lora_b, topk_ids,
           token_lora_ids):
    T, D = hidden_states.shape
    E, two_f, _ = w_gate_up.shape
    F = two_f // 2
    L, _, R, _ = gate_up_lora_a.shape
    K = topk_ids.shape[1]
    TK = T * K
    B = 128
    FC = 1024
    NG = E * L
    NB = TK // B + NG

    i32 = jnp.int32
    tw = topk_weights / jnp.sum(topk_weights, axis=-1, keepdims=True)
    flat_w = tw.reshape(-1)
    flat_e = topk_ids.reshape(-1).astype(i32)
    flat_l = jnp.broadcast_to(token_lora_ids.astype(i32)[:, None],
                              (T, K)).reshape(-1)
    g = flat_e * L + flat_l

    onehot = (g[:, None] == jnp.arange(NG, dtype=i32)[None, :]).astype(i32)
    csum = jnp.cumsum(onehot, axis=0)
    counts = csum[-1]
    rank = jnp.take_along_axis(csum, g[:, None], axis=1)[:, 0] - 1

    bpg = (counts + B - 1) // B
    block_off = jnp.concatenate(
        [jnp.zeros((1,), i32), jnp.cumsum(bpg)[:-1].astype(i32)])
    bids = jnp.arange(NB, dtype=i32)
    bg = jnp.clip(jnp.searchsorted(block_off, bids, side='right').astype(i32)
                  - 1, 0, NG - 1)
    blora = bg % L

    bpe = bpg.reshape(E, L).sum(axis=1)
    base = jnp.concatenate(
        [jnp.zeros((1,), i32), jnp.cumsum(bpe)[:-1].astype(i32)])
    nblk = bpe.astype(i32)

    pslot = (block_off[g] + rank // B) * B + rank % B
    flat_t = jnp.arange(TK, dtype=i32) // K
    rt = jnp.zeros((NB * B,), i32).at[pslot].set(flat_t)
    rw = jnp.zeros((NB * B,), jnp.float32).at[pslot].set(flat_w)
    rw = rw.reshape(NB, B, 1)

    grid_spec = pltpu.PrefetchScalarGridSpec(
        num_scalar_prefetch=4,
        grid=(E,),
        in_specs=[
            pl.BlockSpec((T, D), lambda e, *s: (0, 0)),
            pl.BlockSpec((1, two_f, D), lambda e, *s: (e, 0, 0)),
            pl.BlockSpec((1, D, F), lambda e, *s: (e, 0, 0)),
            pl.BlockSpec((L, 1, R, D), lambda e, *s: (0, e, 0, 0)),
            pl.BlockSpec((L, 1, R, two_f), lambda e, *s: (0, e, 0, 0)),
            pl.BlockSpec((L, 1, R, F), lambda e, *s: (0, e, 0, 0)),
            pl.BlockSpec((L, 1, R, D), lambda e, *s: (0, e, 0, 0)),
            pl.BlockSpec((NB, B, 1), lambda e, *s: (0, 0, 0)),
            pl.BlockSpec((NB, 1, B), lambda e, *s: (0, 0, 0)),
        ],
        out_specs=pl.BlockSpec((T, D), lambda e, *s: (0, 0)),
        scratch_shapes=[
            pltpu.VMEM((B, D), jnp.float32),
        ],
    )
    out = pl.pallas_call(
        functools.partial(_moe_body, B=B, F=F, FC=FC),
        grid_spec=grid_spec,
        out_shape=jax.ShapeDtypeStruct((T, D), jnp.float32),
    )(base, nblk, blora, rt, hidden_states, w_gate_up, w_down,
      gate_up_lora_a, gate_up_lora_b.transpose(0, 1, 3, 2), down_lora_a,
      down_lora_b.transpose(0, 1, 3, 2), rw, rt.reshape(NB, 1, B))
    return out
